# Initial kernel scaffold; baseline (speedup 1.0000x reference)
import jax
import jax.numpy as jnp
from jax.experimental import pallas as pl


def kernel(output, fpn_coord, fpn_diff):
    raise NotImplementedError("write your pallas kernel here")


# speedup vs baseline: 2.4044x; 2.4044x over previous
import jax
import jax.numpy as jnp
from jax import lax
from jax.experimental import pallas as pl
from jax.experimental.pallas import tpu as pltpu
from jax.experimental.pallas import tpu_sc as plsc

B = 8
A = 24
C = 6 * A
D = H = W = 24
K = 128

L = 16
NT = 16
PAIRS = B * K
PP = PAIRS // NT
EPT = PP * 6
NCHUNK_P = PP // L

SPAT = D * H * W
B_STRIDE = C * SPAT
LEVEL_OFF = B * C * SPAT
CH_STRIDE = A * SPAT


def _body(pred_hbm, coord_hbm, gt_hbm, out_hbm,
          coords_v, hidx_v, head_v, base_v, valid_v, idx_v, vals_v, gt_v,
          stage_v, all_v, out_v, shared_v, sem):
    sid = lax.axis_index("s")

    b = sid // (NT // B)
    pair0 = sid * PP

    for c in range(4):
        src = pl.multiple_of(c * PAIRS + pair0, 8)
        pltpu.sync_copy(coord_hbm.at[pl.ds(src, PP)],
                        coords_v.at[pl.ds(c * PP, PP)])
    for ch in range(6):
        src = pl.multiple_of(ch * PAIRS + pair0, 8)
        pltpu.sync_copy(gt_hbm.at[pl.ds(src, PP)],
                        gt_v.at[pl.ds(ch * PP, PP)])
    hidx_v[pl.ds(0, L)] = jnp.full((L,), b * K, jnp.int32)
    pltpu.async_copy(coord_hbm.at[hidx_v], head_v, sem).wait()
    xh = head_v[pl.ds(0, L)]
    anyv = jnp.where(xh > -1, jnp.float32(1.0), jnp.float32(0.0))

    b_off = LEVEL_OFF + b * B_STRIDE

    cnt_acc = jnp.zeros((L,), jnp.float32)
    for c in range(NCHUNK_P):
        o = c * L
        x = coords_v[pl.ds(0 * PP + o, L)]
        y = coords_v[pl.ds(1 * PP + o, L)]
        z = coords_v[pl.ds(2 * PP + o, L)]
        w = coords_v[pl.ds(3 * PP + o, L)]
        validf = jnp.where(x > -1, jnp.float32(1.0), jnp.float32(0.0))
        xc = jnp.maximum(x, 0)
        yc = jnp.maximum(y, 0)
        zc = jnp.maximum(z, 0)
        wc = jnp.maximum(w, 0)
        base = b_off + xc * SPAT + yc * (H * W) + zc * W + wc
        base_v[pl.ds(o, L)] = base
        valid_v[pl.ds(o, L)] = validf
        cnt_acc = cnt_acc + validf

    for ch in range(6):
        for c in range(NCHUNK_P):
            e = ch * PP + c * L
            idx = base_v[pl.ds(c * L, L)] + ch * CH_STRIDE
            idx_v[e // 128, pl.ds(e % 128, L)] = idx

    descs = [pltpu.async_copy(pred_hbm.at[idx_v.at[j]], vals_v.at[j], sem)
             for j in range(3)]
    for dsc in descs:
        dsc.wait()

    lacc = jnp.zeros((L,), jnp.float32)
    for ch in range(6):
        for c in range(NCHUNK_P):
            e = ch * PP + c * L
            pred = vals_v[e // 128, pl.ds(e % 128, L)]
            gt = gt_v[pl.ds(e, L)]
            vm = valid_v[pl.ds(c * L, L)]
            d = pred - gt
            ad = jnp.abs(d)
            loss = jnp.where(ad < 1.0, 0.5 * d * d, ad - 0.5)
            lacc = lacc + loss * vm

    stage_v[0, pl.ds(0, L)] = lacc * anyv
    stage_v[1, pl.ds(0, L)] = cnt_acc * anyv
    pltpu.sync_copy(stage_v, out_hbm.at[sid])


_mesh = plsc.VectorSubcoreMesh(core_axis_name="c", subcore_axis_name="s",
                               num_cores=1)

_kfn = pl.kernel(
    _body,
    out_type=jax.ShapeDtypeStruct((NT, 2, L), jnp.float32),
    mesh=_mesh,
    scratch_types=[
        pltpu.VMEM((4 * PP,), jnp.int32),
        pltpu.VMEM((L,), jnp.int32),
        pltpu.VMEM((L,), jnp.int32),
        pltpu.VMEM((PP,), jnp.int32),
        pltpu.VMEM((PP,), jnp.float32),
        pltpu.VMEM((3, 128), jnp.int32),
        pltpu.VMEM((3, 128), jnp.float32),
        pltpu.VMEM((EPT,), jnp.float32),
        pltpu.VMEM((2, L), jnp.float32),
        pltpu.VMEM((NT, 2, L), jnp.float32),
        pltpu.VMEM((2, L), jnp.float32),
        pltpu.VMEM_SHARED((NT, 2, L), jnp.float32),
        pltpu.SemaphoreType.DMA,
    ],
)


def kernel(output, fpn_coord, fpn_diff):
    pred_flat = output.reshape(-1)
    coord_t = fpn_coord.astype(jnp.int32).reshape(PAIRS, 4).T.reshape(-1)
    gt_t = fpn_diff.astype(jnp.float32).reshape(PAIRS, 6).T.reshape(-1)
    res = _kfn(pred_flat, coord_t, gt_t)
    reg_loss = jnp.sum(res[:, 0, :]).reshape(1)
    reg_weight = jnp.sum(res[:, 1, :]).reshape(1)
    return ([reg_loss], [reg_weight])


# speedup vs baseline: 7.4147x; 3.0838x over previous
import jax
import jax.numpy as jnp
from jax import lax
from jax.experimental import pallas as pl
from jax.experimental.pallas import tpu as pltpu
from jax.experimental.pallas import tpu_sc as plsc

B = 8
A = 24
C = 6 * A
D = H = W = 24
K = 128

L = 16
NT = 16
PAIRS = B * K
PP = PAIRS // NT
EPT = PP * 6
NCHUNK_P = PP // L

B_STRIDE = D * H * W * C
CH_STRIDE = A
LEVEL_OFF = B * B_STRIDE


def _body(pred_hbm, coord_hbm, gt_hbm, out_hbm,
          coords_v, hidx_v, head_v, base_v, valid_v, idx_v, vals_v, gt_v,
          stage_v, all_v, out_v, shared_v, sem):
    sid = lax.axis_index("s")

    b = sid // (NT // B)
    pair0 = sid * PP

    for c in range(4):
        src = pl.multiple_of(c * PAIRS + pair0, 8)
        pltpu.sync_copy(coord_hbm.at[pl.ds(src, PP)],
                        coords_v.at[pl.ds(c * PP, PP)])
    for ch in range(6):
        src = pl.multiple_of(ch * PAIRS + pair0, 8)
        pltpu.sync_copy(gt_hbm.at[pl.ds(src, PP)],
                        gt_v.at[pl.ds(ch * PP, PP)])
    hidx_v[pl.ds(0, L)] = jnp.full((L,), b * K, jnp.int32)
    pltpu.async_copy(coord_hbm.at[hidx_v], head_v, sem).wait()
    xh = head_v[pl.ds(0, L)]
    anyv = jnp.where(xh > -1, jnp.float32(1.0), jnp.float32(0.0))

    b_off = LEVEL_OFF + b * B_STRIDE

    cnt_acc = jnp.zeros((L,), jnp.float32)
    for c in range(NCHUNK_P):
        o = c * L
        x = coords_v[pl.ds(0 * PP + o, L)]
        y = coords_v[pl.ds(1 * PP + o, L)]
        z = coords_v[pl.ds(2 * PP + o, L)]
        w = coords_v[pl.ds(3 * PP + o, L)]
        validf = jnp.where(x > -1, jnp.float32(1.0), jnp.float32(0.0))
        xc = jnp.maximum(x, 0)
        yc = jnp.maximum(y, 0)
        zc = jnp.maximum(z, 0)
        wc = jnp.maximum(w, 0)
        base = b_off + (yc * (H * W) + zc * W + wc) * C + xc
        base_v[pl.ds(o, L)] = base
        valid_v[pl.ds(o, L)] = validf
        cnt_acc = cnt_acc + validf

    for ch in range(6):
        for c in range(NCHUNK_P):
            e = ch * PP + c * L
            idx = base_v[pl.ds(c * L, L)] + ch * CH_STRIDE
            idx_v[e // 128, pl.ds(e % 128, L)] = idx

    descs = [pltpu.async_copy(pred_hbm.at[idx_v.at[j]], vals_v.at[j], sem)
             for j in range(3)]
    for dsc in descs:
        dsc.wait()

    lacc = jnp.zeros((L,), jnp.float32)
    for ch in range(6):
        for c in range(NCHUNK_P):
            e = ch * PP + c * L
            pred = vals_v[e // 128, pl.ds(e % 128, L)]
            gt = gt_v[pl.ds(e, L)]
            vm = valid_v[pl.ds(c * L, L)]
            d = pred - gt
            ad = jnp.abs(d)
            loss = jnp.where(ad < 1.0, 0.5 * d * d, ad - 0.5)
            lacc = lacc + loss * vm

    stage_v[0, pl.ds(0, L)] = lacc * anyv
    stage_v[1, pl.ds(0, L)] = cnt_acc * anyv
    pltpu.sync_copy(stage_v, out_hbm.at[sid])


_mesh = plsc.VectorSubcoreMesh(core_axis_name="c", subcore_axis_name="s",
                               num_cores=1)

_kfn = pl.kernel(
    _body,
    out_type=jax.ShapeDtypeStruct((NT, 2, L), jnp.float32),
    mesh=_mesh,
    scratch_types=[
        pltpu.VMEM((4 * PP,), jnp.int32),
        pltpu.VMEM((L,), jnp.int32),
        pltpu.VMEM((L,), jnp.int32),
        pltpu.VMEM((PP,), jnp.int32),
        pltpu.VMEM((PP,), jnp.float32),
        pltpu.VMEM((3, 128), jnp.int32),
        pltpu.VMEM((3, 128), jnp.float32),
        pltpu.VMEM((EPT,), jnp.float32),
        pltpu.VMEM((2, L), jnp.float32),
        pltpu.VMEM((NT, 2, L), jnp.float32),
        pltpu.VMEM((2, L), jnp.float32),
        pltpu.VMEM_SHARED((NT, 2, L), jnp.float32),
        pltpu.SemaphoreType.DMA,
    ],
)


def kernel(output, fpn_coord, fpn_diff):
    pred_flat = output.transpose(0, 1, 3, 4, 5, 2).reshape(-1)
    coord_t = fpn_coord.astype(jnp.int32).reshape(PAIRS, 4).T.reshape(-1)
    gt_t = fpn_diff.astype(jnp.float32).reshape(PAIRS, 6).T.reshape(-1)
    res = _kfn(pred_flat, coord_t, gt_t)
    reg_loss = jnp.sum(res[:, 0, :]).reshape(1)
    reg_weight = jnp.sum(res[:, 1, :]).reshape(1)
    return ([reg_loss], [reg_weight])


# speedup vs baseline: 14.6127x; 1.9708x over previous
import jax
import jax.numpy as jnp
from jax import lax
from jax.experimental import pallas as pl
from jax.experimental.pallas import tpu as pltpu
from jax.experimental.pallas import tpu_sc as plsc

B = 8
A = 24
C = 6 * A
D = H = W = 24
K = 128

L = 16
NT = 16
PAIRS = B * K
PP = PAIRS // NT
EPT = PP * 6
NCHUNK_P = PP // L

ROWS = 2 * B * D * H * W
B_ROWS = D * H * W
LEVEL_ROWS = B * B_ROWS

TAIL_C0 = 112
TAIL_W = C - TAIL_C0
TAIL_MINOR = W * TAIL_W
TAIL_ROWS = B * D * H * TAIL_MINOR // 128

_CHUNKS = [(ch * A // L, (ch * A + A - 1) // L) for ch in range(5)]


def _smooth_l1(dv):
    ad = jnp.abs(dv)
    return jnp.where(ad < 1.0, 0.5 * dv * dv, ad - 0.5)


def _body(pred_hbm, tail_hbm, coord_hbm, gt_hbm, out_hbm,
          coords_v, head_v, row_v, trow_v, tlane_v, x_v, valid_v,
          rows_v, rows2_v, gt_v, stage_v, sem):
    sid = lax.axis_index("s")
    iota = lax.broadcasted_iota(jnp.int32, (L,), 0)

    b = sid // (NT // B)
    pair0 = sid * PP

    for c in range(4):
        src = pl.multiple_of(c * PAIRS + pair0, 8)
        pltpu.sync_copy(coord_hbm.at[pl.ds(src, PP)],
                        coords_v.at[pl.ds(c * PP, PP)])
    for ch in range(6):
        src = pl.multiple_of(ch * PAIRS + pair0, 8)
        pltpu.sync_copy(gt_hbm.at[pl.ds(src, PP)],
                        gt_v.at[pl.ds(ch * PP, PP)])
    h_start = pl.multiple_of(b * K, 8)
    pltpu.sync_copy(coord_hbm.at[pl.ds(h_start, L)], head_v)
    hv = head_v[pl.ds(0, L)]
    anyv = jnp.where(hv[0] > -1, jnp.float32(1.0), jnp.float32(0.0))

    cnt_acc = jnp.zeros((L,), jnp.float32)
    for c in range(NCHUNK_P):
        o = c * L
        x = coords_v[pl.ds(0 * PP + o, L)]
        y = coords_v[pl.ds(1 * PP + o, L)]
        z = coords_v[pl.ds(2 * PP + o, L)]
        w = coords_v[pl.ds(3 * PP + o, L)]
        validf = jnp.where(x > -1, jnp.float32(1.0), jnp.float32(0.0))
        xc = jnp.maximum(x, 0)
        yc = jnp.maximum(y, 0)
        zc = jnp.maximum(z, 0)
        wc = jnp.maximum(w, 0)
        row = ((LEVEL_ROWS + b * B_ROWS) + yc * (H * W) + zc * W) + wc
        row_v[pl.ds(o, L)] = row
        x_v[pl.ds(o, L)] = xc
        valid_v[pl.ds(o, L)] = validf
        cnt_acc = cnt_acc + validf
        f5 = ((b * (H * W) + yc * W + zc) * TAIL_MINOR
              + wc * TAIL_W + (5 * A + xc - TAIL_C0))
        trow_v[pl.ds(o, L)] = jax.lax.shift_right_logical(f5, 7)
        tlane_v[pl.ds(o, L)] = jax.lax.bitwise_and(f5, 127)

    d0 = pltpu.async_copy(pred_hbm.at[row_v, pl.ds(0, 128)], rows_v, sem)
    d1 = pltpu.async_copy(tail_hbm.at[trow_v], rows2_v, sem)
    d0.wait()
    d1.wait()

    lacc = jnp.zeros((L,), jnp.float32)
    for c in range(NCHUNK_P):
        o = c * L
        xch = x_v[pl.ds(o, L)]
        vch = valid_v[pl.ds(o, L)]
        tlch = tlane_v[pl.ds(o, L)]
        gtch = [gt_v[pl.ds(ch * PP + o, L)] for ch in range(6)]
        for j in range(L):
            p = o + j
            xs = xch[j]
            vf = vch[j]
            for ch in range(5):
                cidx = ch * A + xs
                gts = gtch[ch][j]
                k0, k1 = _CHUNKS[ch]
                for k in (k0, k1):
                    vec = rows_v[p, pl.ds(k * L, L)]
                    m = jnp.where(iota + (k * L) == cidx, vf,
                                  jnp.float32(0.0))
                    lacc = lacc + _smooth_l1(vec - gts) * m
            tl = tlch[j]
            gts = gtch[5][j]
            for k in range(8):
                vec = rows2_v[p, pl.ds(k * L, L)]
                m = jnp.where(iota + (k * L) == tl, vf, jnp.float32(0.0))
                lacc = lacc + _smooth_l1(vec - gts) * m

    stage_v[pl.ds(0, L)] = lacc * anyv
    stage_v[pl.ds(L, L)] = cnt_acc * anyv
    pltpu.sync_copy(stage_v, out_hbm.at[sid])


_mesh = plsc.VectorSubcoreMesh(core_axis_name="c", subcore_axis_name="s",
                               num_cores=1)

_kfn = pl.kernel(
    _body,
    out_type=jax.ShapeDtypeStruct((NT, 128), jnp.float32),
    mesh=_mesh,
    compiler_params=pltpu.CompilerParams(use_tc_tiling_on_sc=True),
    scratch_types=[
        pltpu.VMEM((4 * PP,), jnp.int32),
        pltpu.VMEM((L,), jnp.int32),
        pltpu.VMEM((PP,), jnp.int32),
        pltpu.VMEM((PP,), jnp.int32),
        pltpu.VMEM((PP,), jnp.int32),
        pltpu.VMEM((PP,), jnp.int32),
        pltpu.VMEM((PP,), jnp.float32),
        pltpu.VMEM((PP, 128), jnp.float32),
        pltpu.VMEM((PP, 128), jnp.float32),
        pltpu.VMEM((EPT,), jnp.float32),
        pltpu.VMEM((128,), jnp.float32),
        pltpu.SemaphoreType.DMA,
    ],
)


def kernel(output, fpn_coord, fpn_diff):
    output_t = output.transpose(0, 1, 3, 4, 5, 2)
    pred_rows = output_t.reshape(ROWS, C)
    tail = output_t[1, :, :, :, :, TAIL_C0:].reshape(TAIL_ROWS, 128)
    coord_t = fpn_coord.astype(jnp.int32).reshape(PAIRS, 4).T.reshape(-1)
    gt_t = fpn_diff.astype(jnp.float32).reshape(PAIRS, 6).T.reshape(-1)
    res = _kfn(pred_rows, tail, coord_t, gt_t)
    reg_loss = jnp.sum(res[:, :L]).reshape(1)
    reg_weight = jnp.sum(res[:, L:2 * L]).reshape(1)
    return ([reg_loss], [reg_weight])


# speedup vs baseline: 47.4700x; 3.2485x over previous
import jax
import jax.numpy as jnp
from jax import lax
from jax.experimental import pallas as pl
from jax.experimental.pallas import tpu as pltpu
from jax.experimental.pallas import tpu_sc as plsc

B = 8
A = 24
C = 6 * A
D = H = W = 24
K = 128

L = 16
NT = 16
PAIRS = B * K
PP = PAIRS // NT
EPT = PP * 6
NCHUNK_P = PP // L

ROWS = 2 * B * D * H * W
B_ROWS = D * H * W
LEVEL_ROWS = B * B_ROWS

_CHUNKS = [(ch * A // L, (ch * A + A - 1) // L) for ch in range(5)]


def _smooth_l1(dv):
    ad = jnp.abs(dv)
    return jnp.where(ad < 1.0, 0.5 * dv * dv, ad - 0.5)


def _body(pred_hbm, coord_hbm, gt_hbm, out_hbm,
          coords_v, head_v, row_v, x_v, valid_v,
          rows_v, tail_v, gt_v, stage_v, sem, sem2):
    sid = lax.axis_index("s")
    iota = lax.broadcasted_iota(jnp.int32, (L,), 0)

    b = sid // (NT // B)
    pair0 = sid * PP

    for c in range(4):
        src = pl.multiple_of(c * PAIRS + pair0, 8)
        pltpu.sync_copy(coord_hbm.at[pl.ds(src, PP)],
                        coords_v.at[pl.ds(c * PP, PP)])
    for ch in range(6):
        src = pl.multiple_of(ch * PAIRS + pair0, 8)
        pltpu.sync_copy(gt_hbm.at[pl.ds(src, PP)],
                        gt_v.at[pl.ds(ch * PP, PP)])
    h_start = pl.multiple_of(b * K, 8)
    pltpu.sync_copy(coord_hbm.at[pl.ds(h_start, L)], head_v)
    hv = head_v[pl.ds(0, L)]
    anyv = jnp.where(hv[0] > -1, jnp.float32(1.0), jnp.float32(0.0))

    cnt_acc = jnp.zeros((L,), jnp.float32)
    for c in range(NCHUNK_P):
        o = c * L
        x = coords_v[pl.ds(0 * PP + o, L)]
        y = coords_v[pl.ds(1 * PP + o, L)]
        z = coords_v[pl.ds(2 * PP + o, L)]
        w = coords_v[pl.ds(3 * PP + o, L)]
        validf = jnp.where(x > -1, jnp.float32(1.0), jnp.float32(0.0))
        xc = jnp.maximum(x, 0)
        yc = jnp.maximum(y, 0)
        zc = jnp.maximum(z, 0)
        wc = jnp.maximum(w, 0)
        row = ((LEVEL_ROWS + b * B_ROWS) + yc * (H * W) + zc * W) + wc
        row_v[pl.ds(o, L)] = row
        x_v[pl.ds(o, L)] = xc
        valid_v[pl.ds(o, L)] = validf
        cnt_acc = cnt_acc + validf

    d0 = pltpu.async_copy(pred_hbm.at[row_v, pl.ds(0, 128)], rows_v, sem)
    tails = []
    for c in range(NCHUNK_P):
        rch = row_v[pl.ds(c * L, L)]
        for j in range(L):
            p = c * L + j
            tails.append(pltpu.async_copy(
                pred_hbm.at[rch[j], pl.ds(128, L)],
                tail_v.at[p, pl.ds(0, L)], sem2))
    d0.wait()
    for t in tails:
        t.wait()

    lacc = jnp.zeros((L,), jnp.float32)
    for c in range(NCHUNK_P):
        o = c * L
        xch = x_v[pl.ds(o, L)]
        vch = valid_v[pl.ds(o, L)]
        gtch = [gt_v[pl.ds(ch * PP + o, L)] for ch in range(6)]
        for j in range(L):
            p = o + j
            xs = xch[j]
            vf = vch[j]
            for ch in range(5):
                cidx = ch * A + xs
                gts = gtch[ch][j]
                k0, k1 = _CHUNKS[ch]
                for k in (k0, k1):
                    vec = rows_v[p, pl.ds(k * L, L)]
                    m = jnp.where(iota + (k * L) == cidx, vf,
                                  jnp.float32(0.0))
                    lacc = lacc + _smooth_l1(vec - gts) * m
            cidx = 5 * A + xs
            gts = gtch[5][j]
            vec = rows_v[p, pl.ds(7 * L, L)]
            m = jnp.where(iota + (7 * L) == cidx, vf, jnp.float32(0.0))
            lacc = lacc + _smooth_l1(vec - gts) * m
            vecT = tail_v[p, pl.ds(0, L)]
            mT = jnp.where(iota == cidx - 128, vf, jnp.float32(0.0))
            lacc = lacc + _smooth_l1(vecT - gts) * mT

    stage_v[pl.ds(0, L)] = lacc * anyv
    stage_v[pl.ds(L, L)] = cnt_acc * anyv
    pltpu.sync_copy(stage_v, out_hbm.at[sid])


_mesh = plsc.VectorSubcoreMesh(core_axis_name="c", subcore_axis_name="s",
                               num_cores=1)

_kfn = pl.kernel(
    _body,
    out_type=jax.ShapeDtypeStruct((NT, 128), jnp.float32),
    mesh=_mesh,
    compiler_params=pltpu.CompilerParams(use_tc_tiling_on_sc=True),
    scratch_types=[
        pltpu.VMEM((4 * PP,), jnp.int32),
        pltpu.VMEM((L,), jnp.int32),
        pltpu.VMEM((PP,), jnp.int32),
        pltpu.VMEM((PP,), jnp.int32),
        pltpu.VMEM((PP,), jnp.float32),
        pltpu.VMEM((PP, 128), jnp.float32),
        pltpu.VMEM((PP, 128), jnp.float32),
        pltpu.VMEM((EPT,), jnp.float32),
        pltpu.VMEM((128,), jnp.float32),
        pltpu.SemaphoreType.DMA,
        pltpu.SemaphoreType.DMA,
    ],
)


def kernel(output, fpn_coord, fpn_diff):
    pred_rows = output.transpose(0, 1, 3, 4, 5, 2).reshape(ROWS, C)
    coord_t = fpn_coord.astype(jnp.int32).reshape(PAIRS, 4).T.reshape(-1)
    gt_t = fpn_diff.astype(jnp.float32).reshape(PAIRS, 6).T.reshape(-1)
    res = _kfn(pred_rows, coord_t, gt_t)
    reg_loss = jnp.sum(res[:, :L]).reshape(1)
    reg_weight = jnp.sum(res[:, L:2 * L]).reshape(1)
    return ([reg_loss], [reg_weight])


# speedup vs baseline: 61.8384x; 1.3027x over previous
import jax
import jax.numpy as jnp
from jax import lax
from jax.experimental import pallas as pl
from jax.experimental.pallas import tpu as pltpu
from jax.experimental.pallas import tpu_sc as plsc

B = 8
A = 24
C = 6 * A
D = H = W = 24
K = 128

L = 16
NT = 32
PAIRS = B * K
PP = PAIRS // NT
EPT = PP * 6
NCHUNK_P = PP // L

ROWS = 2 * B * D * H * W
B_ROWS = D * H * W
LEVEL_ROWS = B * B_ROWS

_CHUNKS = [(ch * A // L, (ch * A + A - 1) // L) for ch in range(5)]


def _smooth_l1(dv):
    ad = jnp.abs(dv)
    return jnp.where(ad < 1.0, 0.5 * dv * dv, ad - 0.5)


def _body(pred_hbm, coord_hbm, gt_hbm, out_hbm,
          coords_v, head_v, row_v, x_v, valid_v,
          rows_v, tail_v, gt_v, stage_v, sem, sem2):
    sid = lax.axis_index("s") * 2 + lax.axis_index("c")
    iota = lax.broadcasted_iota(jnp.int32, (L,), 0)

    b = sid // (NT // B)
    pair0 = sid * PP

    for c in range(4):
        src = pl.multiple_of(c * PAIRS + pair0, 8)
        pltpu.sync_copy(coord_hbm.at[pl.ds(src, PP)],
                        coords_v.at[pl.ds(c * PP, PP)])
    h_start = pl.multiple_of(b * K, 8)
    pltpu.sync_copy(coord_hbm.at[pl.ds(h_start, L)], head_v)
    hv = head_v[pl.ds(0, L)]
    anyv = jnp.where(hv[0] > -1, jnp.float32(1.0), jnp.float32(0.0))

    cnt_acc = jnp.zeros((L,), jnp.float32)
    for c in range(NCHUNK_P):
        o = c * L
        x = coords_v[pl.ds(0 * PP + o, L)]
        y = coords_v[pl.ds(1 * PP + o, L)]
        z = coords_v[pl.ds(2 * PP + o, L)]
        w = coords_v[pl.ds(3 * PP + o, L)]
        validf = jnp.where(x > -1, jnp.float32(1.0), jnp.float32(0.0))
        xc = jnp.maximum(x, 0)
        yc = jnp.maximum(y, 0)
        zc = jnp.maximum(z, 0)
        wc = jnp.maximum(w, 0)
        row = ((LEVEL_ROWS + b * B_ROWS) + yc * (H * W) + zc * W) + wc
        row_v[pl.ds(o, L)] = row
        x_v[pl.ds(o, L)] = xc
        valid_v[pl.ds(o, L)] = validf
        cnt_acc = cnt_acc + validf

    d0 = pltpu.async_copy(pred_hbm.at[row_v, pl.ds(0, 128)], rows_v, sem)
    tails = []
    for c in range(NCHUNK_P):
        rch = row_v[pl.ds(c * L, L)]
        for j in range(L):
            p = c * L + j
            tails.append(pltpu.async_copy(
                pred_hbm.at[rch[j], pl.ds(128, L)],
                tail_v.at[p, pl.ds(0, L)], sem2))
    for ch in range(6):
        src = pl.multiple_of(ch * PAIRS + pair0, 8)
        pltpu.sync_copy(gt_hbm.at[pl.ds(src, PP)],
                        gt_v.at[pl.ds(ch * PP, PP)])
    d0.wait()
    for t in tails:
        t.wait()

    lacc = jnp.zeros((L,), jnp.float32)
    for c in range(NCHUNK_P):
        o = c * L
        xch = x_v[pl.ds(o, L)]
        vch = valid_v[pl.ds(o, L)]
        gtch = [gt_v[pl.ds(ch * PP + o, L)] for ch in range(6)]
        for j in range(L):
            p = o + j
            xs = xch[j]
            vf = vch[j]
            for ch in range(5):
                cidx = ch * A + xs
                gts = gtch[ch][j]
                k0, k1 = _CHUNKS[ch]
                v0 = rows_v[p, pl.ds(k0 * L, L)]
                v1 = rows_v[p, pl.ds(k1 * L, L)]
                vec = jnp.where(cidx >= k1 * L, v1, v0)
                lane = jax.lax.bitwise_and(cidx, L - 1)
                m = jnp.where(iota == lane, vf, jnp.float32(0.0))
                lacc = lacc + _smooth_l1(vec - gts) * m
            gts = gtch[5][j]
            v0 = rows_v[p, pl.ds(7 * L, L)]
            v1 = tail_v[p, pl.ds(0, L)]
            vec = jnp.where(xs < 8, v0, v1)
            lane = jnp.where(xs < 8, xs + 8, xs - 8)
            m = jnp.where(iota == lane, vf, jnp.float32(0.0))
            lacc = lacc + _smooth_l1(vec - gts) * m

    stage_v[pl.ds(0, L)] = lacc * anyv
    stage_v[pl.ds(L, L)] = cnt_acc * anyv
    pltpu.sync_copy(stage_v, out_hbm.at[sid])


_mesh = plsc.VectorSubcoreMesh(core_axis_name="c", subcore_axis_name="s")

_kfn = pl.kernel(
    _body,
    out_type=jax.ShapeDtypeStruct((NT, 128), jnp.float32),
    mesh=_mesh,
    compiler_params=pltpu.CompilerParams(use_tc_tiling_on_sc=True),
    scratch_types=[
        pltpu.VMEM((4 * PP,), jnp.int32),
        pltpu.VMEM((L,), jnp.int32),
        pltpu.VMEM((PP,), jnp.int32),
        pltpu.VMEM((PP,), jnp.int32),
        pltpu.VMEM((PP,), jnp.float32),
        pltpu.VMEM((PP, 128), jnp.float32),
        pltpu.VMEM((PP, 128), jnp.float32),
        pltpu.VMEM((EPT,), jnp.float32),
        pltpu.VMEM((128,), jnp.float32),
        pltpu.SemaphoreType.DMA,
        pltpu.SemaphoreType.DMA,
    ],
)


def kernel(output, fpn_coord, fpn_diff):
    pred_rows = output.transpose(0, 1, 3, 4, 5, 2).reshape(ROWS, C)
    coord_t = fpn_coord.astype(jnp.int32).reshape(PAIRS, 4).T.reshape(-1)
    gt_t = fpn_diff.astype(jnp.float32).reshape(PAIRS, 6).T.reshape(-1)
    res = _kfn(pred_rows, coord_t, gt_t)
    reg_loss = jnp.sum(res[:, :L]).reshape(1)
    reg_weight = jnp.sum(res[:, L:2 * L]).reshape(1)
    return ([reg_loss], [reg_weight])


# speedup vs baseline: 61.8549x; 1.0003x over previous
import jax
import jax.numpy as jnp
from jax import lax
from jax.experimental import pallas as pl
from jax.experimental.pallas import tpu as pltpu
from jax.experimental.pallas import tpu_sc as plsc

B = 8
A = 24
C = 6 * A
D = H = W = 24
K = 128

L = 16
NT = 32
PAIRS = B * K
PP = PAIRS // NT
EPT = PP * 6
NCHUNK_P = PP // L

ROWS = 2 * B * D * H * W
B_ROWS = D * H * W
LEVEL_ROWS = B * B_ROWS

_CHUNKS = [(ch * A // L, (ch * A + A - 1) // L) for ch in range(5)]


def _smooth_l1(dv):
    ad = jnp.abs(dv)
    return jnp.where(ad < 1.0, 0.5 * dv * dv, ad - 0.5)


def _body(pred_hbm, coord_hbm, gt_hbm, out_hbm,
          coords_v, head_v, row_v, x_v, valid_v,
          rows_v, tail_v, gt_v, stage_v, sem, sem2):
    sid = lax.axis_index("s") * 2 + lax.axis_index("c")
    iota = lax.broadcasted_iota(jnp.int32, (L,), 0)

    b = sid // (NT // B)
    pair0 = sid * PP
    k0_off = (sid % (NT // B)) * PP

    for c in range(4):
        src = pl.multiple_of((b * 4 + c) * K + k0_off, 8)
        pltpu.sync_copy(coord_hbm.at[pl.ds(src, PP)],
                        coords_v.at[pl.ds(c * PP, PP)])
    h_start = pl.multiple_of(b * 4 * K, 8)
    pltpu.sync_copy(coord_hbm.at[pl.ds(h_start, L)], head_v)
    hv = head_v[pl.ds(0, L)]
    anyv = jnp.where(hv[0] > -1, jnp.float32(1.0), jnp.float32(0.0))

    cnt_acc = jnp.zeros((L,), jnp.float32)
    for c in range(NCHUNK_P):
        o = c * L
        x = coords_v[pl.ds(0 * PP + o, L)]
        y = coords_v[pl.ds(1 * PP + o, L)]
        z = coords_v[pl.ds(2 * PP + o, L)]
        w = coords_v[pl.ds(3 * PP + o, L)]
        validf = jnp.where(x > -1, jnp.float32(1.0), jnp.float32(0.0))
        xc = jnp.maximum(x, 0)
        yc = jnp.maximum(y, 0)
        zc = jnp.maximum(z, 0)
        wc = jnp.maximum(w, 0)
        row = ((LEVEL_ROWS + b * B_ROWS) + yc * (H * W) + zc * W) + wc
        row_v[pl.ds(o, L)] = row
        x_v[pl.ds(o, L)] = xc
        valid_v[pl.ds(o, L)] = validf
        cnt_acc = cnt_acc + validf

    d0 = pltpu.async_copy(pred_hbm.at[row_v, pl.ds(0, 128)], rows_v, sem)
    tails = []
    for c in range(NCHUNK_P):
        rch = row_v[pl.ds(c * L, L)]
        for j in range(L):
            p = c * L + j
            tails.append(pltpu.async_copy(
                pred_hbm.at[rch[j], pl.ds(128, L)],
                tail_v.at[p, pl.ds(0, L)], sem2))
    for ch in range(6):
        src = pl.multiple_of((ch * B + b) * K + k0_off, 8)
        pltpu.sync_copy(gt_hbm.at[pl.ds(src, PP)],
                        gt_v.at[pl.ds(ch * PP, PP)])
    d0.wait()
    for t in tails:
        t.wait()

    lacc = jnp.zeros((L,), jnp.float32)
    for c in range(NCHUNK_P):
        o = c * L
        xch = x_v[pl.ds(o, L)]
        vch = valid_v[pl.ds(o, L)]
        gtch = [gt_v[pl.ds(ch * PP + o, L)] for ch in range(6)]
        for j in range(L):
            p = o + j
            xs = xch[j]
            vf = vch[j]
            for ch in range(5):
                cidx = ch * A + xs
                gts = gtch[ch][j]
                k0, k1 = _CHUNKS[ch]
                v0 = rows_v[p, pl.ds(k0 * L, L)]
                v1 = rows_v[p, pl.ds(k1 * L, L)]
                vec = jnp.where(cidx >= k1 * L, v1, v0)
                lane = jax.lax.bitwise_and(cidx, L - 1)
                m = jnp.where(iota == lane, vf, jnp.float32(0.0))
                lacc = lacc + _smooth_l1(vec - gts) * m
            gts = gtch[5][j]
            v0 = rows_v[p, pl.ds(7 * L, L)]
            v1 = tail_v[p, pl.ds(0, L)]
            vec = jnp.where(xs < 8, v0, v1)
            lane = jnp.where(xs < 8, xs + 8, xs - 8)
            m = jnp.where(iota == lane, vf, jnp.float32(0.0))
            lacc = lacc + _smooth_l1(vec - gts) * m

    stage_v[pl.ds(0, L)] = lacc * anyv
    stage_v[pl.ds(L, L)] = cnt_acc * anyv
    pltpu.sync_copy(stage_v, out_hbm.at[sid])


_mesh = plsc.VectorSubcoreMesh(core_axis_name="c", subcore_axis_name="s")

_kfn = pl.kernel(
    _body,
    out_type=jax.ShapeDtypeStruct((NT, 128), jnp.float32),
    mesh=_mesh,
    compiler_params=pltpu.CompilerParams(use_tc_tiling_on_sc=True),
    scratch_types=[
        pltpu.VMEM((4 * PP,), jnp.int32),
        pltpu.VMEM((L,), jnp.int32),
        pltpu.VMEM((PP,), jnp.int32),
        pltpu.VMEM((PP,), jnp.int32),
        pltpu.VMEM((PP,), jnp.float32),
        pltpu.VMEM((PP, 128), jnp.float32),
        pltpu.VMEM((PP, 128), jnp.float32),
        pltpu.VMEM((EPT,), jnp.float32),
        pltpu.VMEM((128,), jnp.float32),
        pltpu.SemaphoreType.DMA,
        pltpu.SemaphoreType.DMA,
    ],
)


def kernel(output, fpn_coord, fpn_diff):
    pred_rows = output.transpose(0, 1, 3, 4, 5, 2).reshape(ROWS, C)
    coord_t = fpn_coord.astype(jnp.int32).transpose(0, 1, 3, 2).reshape(-1)
    gt_t = fpn_diff.astype(jnp.float32).transpose(0, 3, 1, 2).reshape(-1)
    res = _kfn(pred_rows, coord_t, gt_t)
    reg_loss = jnp.sum(res[:, :L]).reshape(1)
    reg_weight = jnp.sum(res[:, L:2 * L]).reshape(1)
    return ([reg_loss], [reg_weight])


# speedup vs baseline: 69.6340x; 1.1258x over previous
import jax
import jax.numpy as jnp
from jax import lax
from jax.experimental import pallas as pl
from jax.experimental.pallas import tpu as pltpu
from jax.experimental.pallas import tpu_sc as plsc

B = 8
A = 24
C = 6 * A
D = H = W = 24
K = 128

L = 16
NT = 32
PAIRS = B * K
PP = PAIRS // NT
EPT = PP * 6
NCHUNK_P = PP // L

ROWS = 2 * B * D * H * W
B_ROWS = D * H * W
LEVEL_ROWS = B * B_ROWS

_CHUNKS = [(ch * A // L, (ch * A + A - 1) // L) for ch in range(5)]


def _smooth_l1(dv):
    ad = jnp.abs(dv)
    return jnp.where(ad < 1.0, 0.5 * dv * dv, ad - 0.5)


def _body(pred_hbm, coord_hbm, gt_hbm, out_hbm,
          coords_v, row_v, x_v, valid_v,
          rows_v, tail_v, gt_v, stage_v, sem, sem2, sem3):
    sid = lax.axis_index("s") * 2 + lax.axis_index("c")
    iota = lax.broadcasted_iota(jnp.int32, (L,), 0)

    b = sid // (NT // B)
    pair0 = sid * PP
    k0_off = (sid % (NT // B)) * PP

    src = pl.multiple_of(b * 4 * K, 8)
    pltpu.sync_copy(coord_hbm.at[pl.ds(src, 4 * K)], coords_v)
    hv = coords_v[pl.ds(0, L)]
    anyv = jnp.where(hv[0] > -1, jnp.float32(1.0), jnp.float32(0.0))

    cnt_acc = jnp.zeros((L,), jnp.float32)
    for c in range(NCHUNK_P):
        o = pl.multiple_of(k0_off + c * L, L)
        x = coords_v[pl.ds(0 * K + o, L)]
        y = coords_v[pl.ds(1 * K + o, L)]
        z = coords_v[pl.ds(2 * K + o, L)]
        w = coords_v[pl.ds(3 * K + o, L)]
        validf = jnp.where(x > -1, jnp.float32(1.0), jnp.float32(0.0))
        xc = jnp.maximum(x, 0)
        yc = jnp.maximum(y, 0)
        zc = jnp.maximum(z, 0)
        wc = jnp.maximum(w, 0)
        row = ((LEVEL_ROWS + b * B_ROWS) + yc * (H * W) + zc * W) + wc
        row_v[pl.ds(c * L, L)] = row
        x_v[pl.ds(c * L, L)] = xc
        valid_v[pl.ds(c * L, L)] = validf
        cnt_acc = cnt_acc + validf

    d0 = pltpu.async_copy(pred_hbm.at[row_v, pl.ds(0, 128)], rows_v, sem)
    tails = []
    for c in range(NCHUNK_P):
        rch = row_v[pl.ds(c * L, L)]
        for j in range(L):
            p = c * L + j
            tails.append(pltpu.async_copy(
                pred_hbm.at[rch[j], pl.ds(128, L)],
                tail_v.at[p, pl.ds(0, L)], sem2))
    gts_d = []
    for ch in range(6):
        src = pl.multiple_of((ch * B + b) * K + k0_off, 8)
        gts_d.append(pltpu.async_copy(gt_hbm.at[pl.ds(src, PP)],
                                      gt_v.at[pl.ds(ch * PP, PP)], sem3))
    d0.wait()
    for t in tails:
        t.wait()
    for t in gts_d:
        t.wait()

    lacc = jnp.zeros((L,), jnp.float32)
    for c in range(NCHUNK_P):
        o = c * L
        xch = x_v[pl.ds(o, L)]
        vch = valid_v[pl.ds(o, L)]
        gtch = [gt_v[pl.ds(ch * PP + o, L)] for ch in range(6)]
        for j in range(L):
            p = o + j
            xs = xch[j]
            vf = vch[j]
            for ch in range(5):
                cidx = ch * A + xs
                gts = gtch[ch][j]
                k0, k1 = _CHUNKS[ch]
                v0 = rows_v[p, pl.ds(k0 * L, L)]
                v1 = rows_v[p, pl.ds(k1 * L, L)]
                vec = jnp.where(cidx >= k1 * L, v1, v0)
                lane = jax.lax.bitwise_and(cidx, L - 1)
                m = jnp.where(iota == lane, vf, jnp.float32(0.0))
                lacc = lacc + _smooth_l1(vec - gts) * m
            gts = gtch[5][j]
            v0 = rows_v[p, pl.ds(7 * L, L)]
            v1 = tail_v[p, pl.ds(0, L)]
            vec = jnp.where(xs < 8, v0, v1)
            lane = jnp.where(xs < 8, xs + 8, xs - 8)
            m = jnp.where(iota == lane, vf, jnp.float32(0.0))
            lacc = lacc + _smooth_l1(vec - gts) * m

    stage_v[pl.ds(0, L)] = lacc * anyv
    stage_v[pl.ds(L, L)] = cnt_acc * anyv
    pltpu.sync_copy(stage_v, out_hbm.at[sid])


_mesh = plsc.VectorSubcoreMesh(core_axis_name="c", subcore_axis_name="s")

_kfn = pl.kernel(
    _body,
    out_type=jax.ShapeDtypeStruct((NT, 128), jnp.float32),
    mesh=_mesh,
    compiler_params=pltpu.CompilerParams(use_tc_tiling_on_sc=True),
    scratch_types=[
        pltpu.VMEM((4 * K,), jnp.int32),
        pltpu.VMEM((PP,), jnp.int32),
        pltpu.VMEM((PP,), jnp.int32),
        pltpu.VMEM((PP,), jnp.float32),
        pltpu.VMEM((PP, 128), jnp.float32),
        pltpu.VMEM((PP, 128), jnp.float32),
        pltpu.VMEM((EPT,), jnp.float32),
        pltpu.VMEM((128,), jnp.float32),
        pltpu.SemaphoreType.DMA,
        pltpu.SemaphoreType.DMA,
        pltpu.SemaphoreType.DMA,
    ],
)


def kernel(output, fpn_coord, fpn_diff):
    pred_rows = output.transpose(0, 1, 3, 4, 5, 2).reshape(ROWS, C)
    coord_t = fpn_coord.astype(jnp.int32).transpose(0, 1, 3, 2).reshape(-1)
    gt_t = fpn_diff.astype(jnp.float32).transpose(0, 3, 1, 2).reshape(-1)
    res = _kfn(pred_rows, coord_t, gt_t)
    reg_loss = jnp.sum(res[:, :L]).reshape(1)
    reg_weight = jnp.sum(res[:, L:2 * L]).reshape(1)
    return ([reg_loss], [reg_weight])


# speedup vs baseline: 69.6689x; 1.0005x over previous
import jax
import jax.numpy as jnp
from jax import lax
from jax.experimental import pallas as pl
from jax.experimental.pallas import tpu as pltpu
from jax.experimental.pallas import tpu_sc as plsc

B = 8
A = 24
C = 6 * A
D = H = W = 24
K = 128

L = 16
NT = 32
PAIRS = B * K
PP = PAIRS // NT
EPT = PP * 6
NCHUNK_P = PP // L

ROWS = 2 * B * D * H * W
B_ROWS = D * H * W
LEVEL_ROWS = B * B_ROWS

_CHUNKS = [(ch * A // L, (ch * A + A - 1) // L) for ch in range(5)]


def _smooth_l1(dv):
    ad = jnp.abs(dv)
    return jnp.where(ad < 1.0, 0.5 * dv * dv, ad - 0.5)


def _body(pred_hbm, coord_hbm, gt_hbm, out_hbm,
          coords_v, row_v, x_v, valid_v,
          rows_v, tail_v, gt_v, stage_v, sem, sem2, sem3):
    sid = lax.axis_index("s") * 2 + lax.axis_index("c")
    iota = lax.broadcasted_iota(jnp.int32, (L,), 0)

    b = sid // (NT // B)
    k0_off = (sid % (NT // B)) * PP

    src = pl.multiple_of(b * 4 * K, 8)
    pltpu.sync_copy(coord_hbm.at[pl.ds(src, 4 * K)], coords_v)
    hv = coords_v[pl.ds(0, L)]
    anyv = jnp.where(hv[0] > -1, jnp.float32(1.0), jnp.float32(0.0))

    cnt_acc = jnp.zeros((L,), jnp.float32)
    for c in range(NCHUNK_P):
        o = pl.multiple_of(k0_off + c * L, L)
        x = coords_v[pl.ds(0 * K + o, L)]
        y = coords_v[pl.ds(1 * K + o, L)]
        z = coords_v[pl.ds(2 * K + o, L)]
        w = coords_v[pl.ds(3 * K + o, L)]
        validf = jnp.where(x > -1, jnp.float32(1.0), jnp.float32(0.0))
        xc = jnp.maximum(x, 0)
        yc = jnp.maximum(y, 0)
        zc = jnp.maximum(z, 0)
        wc = jnp.maximum(w, 0)
        row = ((LEVEL_ROWS + b * B_ROWS) + yc * (H * W) + zc * W) + wc
        row_v[pl.ds(c * L, L)] = row
        x_v[pl.ds(c * L, L)] = xc
        valid_v[pl.ds(c * L, L)] = validf
        cnt_acc = cnt_acc + validf

    d0 = pltpu.async_copy(pred_hbm.at[row_v, pl.ds(0, 128)], rows_v, sem)
    tails = []
    for c in range(NCHUNK_P):
        rch = row_v[pl.ds(c * L, L)]
        for j in range(L):
            p = c * L + j
            tails.append(pltpu.async_copy(
                pred_hbm.at[rch[j], pl.ds(128, L)],
                tail_v.at[p, pl.ds(0, L)], sem2))
    gts_d = []
    for ch in range(6):
        src = pl.multiple_of((ch * B + b) * K + k0_off, 8)
        gts_d.append(pltpu.async_copy(gt_hbm.at[pl.ds(src, PP)],
                                      gt_v.at[pl.ds(ch * PP, PP)], sem3))
    d0.wait()
    for t in tails:
        t.wait()
    for t in gts_d:
        t.wait()

    lacc = jnp.zeros((L,), jnp.float32)
    for c in range(NCHUNK_P):
        o = c * L
        xch = x_v[pl.ds(o, L)]
        vch = valid_v[pl.ds(o, L)]
        gtch = [gt_v[pl.ds(ch * PP + o, L)] for ch in range(6)]
        for j in range(L):
            p = o + j
            xs = xch[j]
            vf = vch[j]
            for ch in range(5):
                cidx = ch * A + xs
                gts = gtch[ch][j]
                k0, k1 = _CHUNKS[ch]
                v0 = rows_v[p, pl.ds(k0 * L, L)]
                v1 = rows_v[p, pl.ds(k1 * L, L)]
                vec = jnp.where(cidx >= k1 * L, v1, v0)
                lane = jax.lax.bitwise_and(cidx, L - 1)
                m = jnp.where(iota == lane, vf, jnp.float32(0.0))
                lacc = lacc + _smooth_l1(vec - gts) * m
            gts = gtch[5][j]
            v0 = rows_v[p, pl.ds(7 * L, L)]
            v1 = tail_v[p, pl.ds(0, L)]
            vec = jnp.where(xs < 8, v0, v1)
            lane = jnp.where(xs < 8, xs + 8, xs - 8)
            m = jnp.where(iota == lane, vf, jnp.float32(0.0))
            lacc = lacc + _smooth_l1(vec - gts) * m

    stage_v[pl.ds(0, L)] = lacc * anyv
    stage_v[pl.ds(L, L)] = cnt_acc * anyv
    pltpu.sync_copy(stage_v, out_hbm.at[sid])


_mesh = plsc.VectorSubcoreMesh(core_axis_name="c", subcore_axis_name="s")

_kfn = pl.kernel(
    _body,
    out_type=jax.ShapeDtypeStruct((NT, 128), jnp.float32),
    mesh=_mesh,
    compiler_params=pltpu.CompilerParams(use_tc_tiling_on_sc=True),
    scratch_types=[
        pltpu.VMEM((4 * K,), jnp.int32),
        pltpu.VMEM((PP,), jnp.int32),
        pltpu.VMEM((PP,), jnp.int32),
        pltpu.VMEM((PP,), jnp.float32),
        pltpu.VMEM((PP, 128), jnp.float32),
        pltpu.VMEM((PP, 128), jnp.float32),
        pltpu.VMEM((EPT,), jnp.float32),
        pltpu.VMEM((128,), jnp.float32),
        pltpu.SemaphoreType.DMA,
        pltpu.SemaphoreType.DMA,
        pltpu.SemaphoreType.DMA,
    ],
)


def kernel(output, fpn_coord, fpn_diff):
    pred_rows = output.transpose(0, 1, 3, 4, 5, 2).reshape(ROWS, C)
    coord_t = fpn_coord.astype(jnp.int32).transpose(0, 1, 3, 2).reshape(-1)
    gt_t = fpn_diff.astype(jnp.float32).transpose(0, 3, 1, 2).reshape(-1)
    res = _kfn(pred_rows, coord_t, gt_t)
    reg_loss = jnp.sum(res[:, :L]).reshape(1)
    reg_weight = jnp.sum(res[:, L:2 * L]).reshape(1)
    return ([reg_loss], [reg_weight])
